# baseline (speedup 1.0000x reference)
import functools

import jax
import jax.numpy as jnp
from jax import lax
from jax.experimental import pallas as pl
from jax.experimental.pallas import tpu as pltpu
from jax.experimental.pallas import tpu_sc as plsc

BATCH = 4096
SEQ_LEN = 50
NUM_RESIDUES = 1000
EMBED_DIM = 64

NUM_WORKERS = 32
TOTAL = BATCH * SEQ_LEN
PER_W = TOTAL // NUM_WORKERS
BATCH_PER_W = BATCH // NUM_WORKERS
NBUF = 4
GROUPS = 16
GR = PER_W // GROUPS
GB = BATCH_PER_W // GROUPS


def _sc_gather(idx_flat, table):
    mesh = plsc.VectorSubcoreMesh(core_axis_name="c", subcore_axis_name="s")

    @functools.partial(
        pl.kernel,
        mesh=mesh,
        compiler_params=pltpu.CompilerParams(use_tc_tiling_on_sc=False),
        out_type=jax.ShapeDtypeStruct((BATCH, SEQ_LEN, EMBED_DIM), jnp.float32),
        scratch_types=[
            pltpu.VMEM((PER_W,), jnp.int32),
            pltpu.VMEM((NBUF, GR, EMBED_DIM), jnp.float32),
        ]
        + [pltpu.SemaphoreType.DMA] * (2 * NBUF),
    )
    def k(idx_hbm, table_hbm, out_hbm, idx_v, rows_v, *sems):
        gsem, osem = sems[:NBUF], sems[NBUF:]
        wid = lax.axis_index("s") * 2 + lax.axis_index("c")
        base = wid * PER_W
        bbase = wid * BATCH_PER_W
        pltpu.sync_copy(idx_hbm.at[pl.ds(base, PER_W)], idx_v)

        def fire_gather(g, b):
            pltpu.async_copy(
                table_hbm.at[idx_v.at[pl.ds(g * GR, GR)]], rows_v.at[b], gsem[b]
            )

        def wait_gather(b):
            pltpu.make_async_copy(
                table_hbm.at[pl.ds(0, GR)], rows_v.at[b], gsem[b]
            ).wait()

        def fire_scatters(g, b):
            for r in range(GB):
                pltpu.async_copy(
                    rows_v.at[b].at[pl.ds(r * SEQ_LEN, SEQ_LEN)],
                    out_hbm.at[bbase + g * GB + r],
                    osem[b],
                )

        def wait_scatters(b):
            for r in range(GB):
                pltpu.make_async_copy(
                    rows_v.at[b].at[pl.ds(0, SEQ_LEN)],
                    out_hbm.at[bbase],
                    osem[b],
                ).wait()

        for b in range(NBUF):
            fire_gather(b, b)

        def body(i, carry):
            g0 = i * NBUF
            for b in range(NBUF):
                g = g0 + b
                wait_gather(b)
                fire_scatters(g, b)
                wait_scatters(b)
                fire_gather(g + NBUF, b)
            return carry

        lax.fori_loop(0, GROUPS // NBUF - 1, body, 0)

        for b in range(NBUF):
            g = GROUPS - NBUF + b
            wait_gather(b)
            fire_scatters(g, b)
        for b in range(NBUF):
            wait_scatters(b)

    return k(idx_flat, table)


def kernel(indices, embeddings):
    idx = jnp.clip(indices, 0, NUM_RESIDUES - 1)
    return _sc_gather(idx.reshape(TOTAL), embeddings)
